# Initial kernel scaffold; baseline (speedup 1.0000x reference)
import functools

import jax
import jax.numpy as jnp
from jax import lax
from jax.experimental import pallas as pl
from jax.experimental.pallas import tpu as pltpu
from jax.experimental.pallas import tpu_sc as plsc

N_LEVELS = 16
TABLE_SIZE = 1 << 19
N_POINTS = 1048576
RES = [16 << i for i in range(N_LEVELS)]

N_CORES = 2
N_SUBCORES = 16
NW = N_CORES * N_SUBCORES
LANES = 16

COARSE = list(range(11))
FINE = list(range(11, 16))

C = 512
GROUP = 128
G = C // GROUP
P = N_POINTS // NW
NCHUNK = P // C

CBASE = []
_off = 0
for _l in COARSE:
    CBASE.append(_off)
    _off += 2 * (RES[_l] + 1)
COARSE_WORDS = _off

PBASE = []
_off = 0
for _l in FINE:
    PBASE.append(_off)
    _off += RES[_l]
PAIR_ROWS = _off

_mesh = plsc.VectorSubcoreMesh(core_axis_name="c", subcore_axis_name="s")


@functools.partial(
    pl.kernel,
    mesh=_mesh,
    out_type=jax.ShapeDtypeStruct((N_POINTS, 2 * N_LEVELS), jnp.float32),
    scratch_types=[
        pltpu.VMEM((COARSE_WORDS,), jnp.float32),
        pltpu.VMEM((C,), jnp.float32),
        pltpu.VMEM((len(FINE), C), jnp.int32),
        pltpu.VMEM((len(FINE), C, 4), jnp.float32),
        pltpu.VMEM((C, 2 * N_LEVELS), jnp.float32),
        pltpu.SemaphoreType.DMA,
    ],
)
def _sc_embed(x_hbm, coarse_hbm, pair_hbm, out_hbm,
              coarse_v, x_v, idx_v, dest_v, out_v, sem):
    wid = lax.axis_index("s") * N_CORES + lax.axis_index("c")
    pltpu.sync_copy(coarse_hbm, coarse_v)
    base_pt = wid * P

    @pl.loop(0, NCHUNK)
    def _chunk(ci):
        start = base_pt + ci * C
        pltpu.sync_copy(x_hbm.at[pl.ds(start, C)], x_v)

        iota = lax.iota(jnp.int32, LANES)

        copies = []
        for li, l in enumerate(FINE):
            r = float(RES[l])
            pb = PBASE[li]

            @pl.loop(0, C, step=LANES)
            def _idx(p, li=li, r=r, pb=pb):
                xv = x_v[pl.ds(p, LANES)]
                bi = (xv * r).astype(jnp.int32)
                idx_v[li, pl.ds(p, LANES)] = bi + pb

            for g in range(G):
                copies.append(pltpu.async_copy(
                    pair_hbm.at[idx_v.at[li, pl.ds(g * GROUP, GROUP)]],
                    dest_v.at[li, pl.ds(g * GROUP, GROUP)],
                    sem))

        for li, l in enumerate(COARSE):
            r = float(RES[l])
            gs = 1.0 / RES[l]
            cb = CBASE[li]

            @pl.loop(0, C, step=LANES)
            def _coarse(p, r=r, gs=gs, cb=cb, col=2 * l):
                xv = x_v[pl.ds(p, LANES)]
                bi = (xv * r).astype(jnp.int32)
                w = (xv - bi.astype(jnp.float32) * gs) * r
                flat = bi * 2 + cb
                t00 = plsc.load_gather(coarse_v, [flat])
                t01 = plsc.load_gather(coarse_v, [flat + 1])
                t10 = plsc.load_gather(coarse_v, [flat + 2])
                t11 = plsc.load_gather(coarse_v, [flat + 3])
                omw = 1.0 - w
                rows = p + iota
                plsc.store_scatter(
                    out_v, [rows, jnp.full((LANES,), col, jnp.int32)],
                    t00 * omw + t10 * w)
                plsc.store_scatter(
                    out_v, [rows, jnp.full((LANES,), col + 1, jnp.int32)],
                    t01 * omw + t11 * w)

        for cp in copies:
            cp.wait()

        for li, l in enumerate(FINE):
            r = float(RES[l])
            gs = 1.0 / RES[l]

            @pl.loop(0, C, step=LANES)
            def _fine(p, li=li, r=r, gs=gs, col=2 * l):
                xv = x_v[pl.ds(p, LANES)]
                bi = (xv * r).astype(jnp.int32)
                w = (xv - bi.astype(jnp.float32) * gs) * r
                rows = p + iota
                lsp = jnp.full((LANES,), li, jnp.int32)
                t00 = plsc.load_gather(dest_v, [lsp, rows, jnp.full((LANES,), 0, jnp.int32)])
                t01 = plsc.load_gather(dest_v, [lsp, rows, jnp.full((LANES,), 1, jnp.int32)])
                t10 = plsc.load_gather(dest_v, [lsp, rows, jnp.full((LANES,), 2, jnp.int32)])
                t11 = plsc.load_gather(dest_v, [lsp, rows, jnp.full((LANES,), 3, jnp.int32)])
                omw = 1.0 - w
                plsc.store_scatter(
                    out_v, [rows, jnp.full((LANES,), col, jnp.int32)],
                    t00 * omw + t10 * w)
                plsc.store_scatter(
                    out_v, [rows, jnp.full((LANES,), col + 1, jnp.int32)],
                    t01 * omw + t11 * w)

        pltpu.sync_copy(out_v, out_hbm.at[pl.ds(start, C)])


def kernel(x, tables):
    xf = x.reshape(N_POINTS)
    coarse_blob = jnp.concatenate(
        [tables[l, :RES[l] + 1].reshape(-1) for l in COARSE], axis=0)
    parts = []
    for l in FINE:
        t = tables[l]
        r = RES[l]
        if r == TABLE_SIZE:
            nxt = jnp.concatenate([t[1:], t[:1]], axis=0)
        else:
            nxt = t[1:r + 1]
        parts.append(jnp.concatenate([t[:r], nxt], axis=1))
    pair = jnp.concatenate(parts, axis=0)
    return _sc_embed(xf, coarse_blob, pair)


# speedup vs baseline: 88.0521x; 88.0521x over previous
import dataclasses
import functools

import jax
import jax.numpy as jnp
from jax import lax
from jax.experimental import pallas as pl
from jax.experimental.pallas import tpu as pltpu
from jax.experimental.pallas import tpu_sc as plsc

N_LEVELS = 16
TABLE_SIZE = 1 << 19
N_POINTS = 1048576
RES = [16 << i for i in range(N_LEVELS)]

N_CORES = 2
N_SUBCORES = 16
NW = N_CORES * N_SUBCORES
LANES = 16

COARSE = list(range(11))
FINE = list(range(11, 16))

C = 512
GROUP = 128
G = C // GROUP
P = N_POINTS // NW
NCHUNK = P // C

CBASE = []
_off = 0
for _l in COARSE:
    CBASE.append(_off)
    _off += 2 * (RES[_l] + 1)
COARSE_WORDS = _off

PBASE = []
_off = 0
for _l in FINE:
    PBASE.append(_off)
    _off += RES[_l]
PAIR_ROWS = _off

_mesh = plsc.VectorSubcoreMesh(core_axis_name="c", subcore_axis_name="s")

_cp = pltpu.CompilerParams()
if "needs_layout_passes" in pltpu.CompilerParams.__dataclass_fields__:
    _cp = dataclasses.replace(_cp, needs_layout_passes=False)
if "use_tc_tiling_on_sc" in pltpu.CompilerParams.__dataclass_fields__:
    _cp = dataclasses.replace(_cp, use_tc_tiling_on_sc=False)


@functools.partial(
    pl.kernel,
    mesh=_mesh,
    compiler_params=_cp,
    out_type=jax.ShapeDtypeStruct((N_POINTS, 2 * N_LEVELS), jnp.float32),
    scratch_types=(
        [pltpu.VMEM((COARSE_WORDS,), jnp.float32),
         pltpu.VMEM((C,), jnp.float32)]
        + [pltpu.VMEM((2 * GROUP,), jnp.int32) for _ in FINE for _ in range(G)]
        + [pltpu.VMEM((2 * GROUP, 4), jnp.float32) for _ in FINE for _ in range(G)]
        + [pltpu.VMEM((C, 2 * N_LEVELS), jnp.float32),
           pltpu.SemaphoreType.DMA]
    ),
)
def _sc_embed(x_hbm, coarse_hbm, pair_hbm, out_hbm,
              coarse_v, x_v, *rest):
    nf = len(FINE) * G
    idx_refs = rest[:nf]
    dest_refs = rest[nf:2 * nf]
    out_v = rest[2 * nf]
    sem = rest[2 * nf + 1]
    wid = lax.axis_index("s") * N_CORES + lax.axis_index("c")
    pltpu.sync_copy(coarse_hbm, coarse_v)
    base_pt = wid * P

    @pl.loop(0, NCHUNK)
    def _chunk(ci):
        start = base_pt + ci * C
        pltpu.sync_copy(x_hbm.at[pl.ds(start, C)], x_v)

        iota = lax.iota(jnp.int32, LANES)

        copies = []
        for li, l in enumerate(FINE):
            r = float(RES[l])
            pb = PBASE[li]

            for g in range(G):
                idx_ref = idx_refs[li * G + g]

                @pl.loop(0, GROUP, step=LANES)
                def _idx(p, idx_ref=idx_ref, r=r, pb=pb, g=g):
                    xv = x_v[pl.ds(g * GROUP + p, LANES)]
                    bi = (xv * r).astype(jnp.int32)
                    plsc.store_scatter(idx_ref, [2 * p + 2 * iota],
                                       (bi + pb) * 2)

                copies.append(pltpu.async_copy(
                    pair_hbm.at[idx_ref],
                    dest_refs[li * G + g],
                    sem))

        for li, l in enumerate(COARSE):
            r = float(RES[l])
            gs = 1.0 / RES[l]
            cb = CBASE[li]

            @pl.loop(0, C, step=LANES)
            def _coarse(p, r=r, gs=gs, cb=cb, col=2 * l):
                xv = x_v[pl.ds(p, LANES)]
                bi = (xv * r).astype(jnp.int32)
                w = (xv - bi.astype(jnp.float32) * gs) * r
                flat = bi * 2 + cb
                t00 = plsc.load_gather(coarse_v, [flat])
                t01 = plsc.load_gather(coarse_v, [flat + 1])
                t10 = plsc.load_gather(coarse_v, [flat + 2])
                t11 = plsc.load_gather(coarse_v, [flat + 3])
                omw = 1.0 - w
                rows = p + iota
                plsc.store_scatter(
                    out_v, [rows, jnp.full((LANES,), col, jnp.int32)],
                    t00 * omw + t10 * w)
                plsc.store_scatter(
                    out_v, [rows, jnp.full((LANES,), col + 1, jnp.int32)],
                    t01 * omw + t11 * w)

        for cp_ in copies:
            cp_.wait()

        for li, l in enumerate(FINE):
            r = float(RES[l])
            gs = 1.0 / RES[l]

            for g in range(G):
                dest_ref = dest_refs[li * G + g]

                @pl.loop(0, GROUP, step=LANES)
                def _fine(p, dest_ref=dest_ref, r=r, gs=gs, col=2 * l, g=g):
                    xv = x_v[pl.ds(g * GROUP + p, LANES)]
                    bi = (xv * r).astype(jnp.int32)
                    w = (xv - bi.astype(jnp.float32) * gs) * r
                    rows = p + iota
                    t00 = plsc.load_gather(dest_ref, [rows, jnp.full((LANES,), 0, jnp.int32)])
                    t01 = plsc.load_gather(dest_ref, [rows, jnp.full((LANES,), 1, jnp.int32)])
                    t10 = plsc.load_gather(dest_ref, [rows, jnp.full((LANES,), 2, jnp.int32)])
                    t11 = plsc.load_gather(dest_ref, [rows, jnp.full((LANES,), 3, jnp.int32)])
                    omw = 1.0 - w
                    orow = g * GROUP + p + iota
                    plsc.store_scatter(
                        out_v, [orow, jnp.full((LANES,), col, jnp.int32)],
                        t00 * omw + t10 * w)
                    plsc.store_scatter(
                        out_v, [orow, jnp.full((LANES,), col + 1, jnp.int32)],
                        t01 * omw + t11 * w)

        pltpu.sync_copy(out_v, out_hbm.at[pl.ds(start, C)])


def kernel(x, tables):
    xf = x.reshape(N_POINTS)
    coarse_blob = jnp.concatenate(
        [tables[l, :RES[l] + 1].reshape(-1) for l in COARSE], axis=0)
    parts = []
    for l in FINE:
        t = tables[l]
        r = RES[l]
        if r == TABLE_SIZE:
            nxt = jnp.concatenate([t[1:], t[:1]], axis=0)
        else:
            nxt = t[1:r + 1]
        parts.append(jnp.concatenate([t[:r], nxt], axis=1))
    pair = jnp.concatenate(parts, axis=0)
    return _sc_embed(xf, coarse_blob, pair)
